# baseline (speedup 1.0000x reference)
import functools

import jax
import jax.numpy as jnp
from jax import lax
from jax.experimental import pallas as pl
from jax.experimental.pallas import tpu as pltpu
from jax.experimental.pallas import tpu_sc as plsc


_RB = 2048
_CH = 16


def _edges_mask_body(N, Ed, B, nfull, ptr_ref, in_ref, clscol_ref, clsslab_ref,
                     out_e_ref, out_m_ref, out_nb_ref, out_cm_ref):
    b = pl.program_id(0)
    r = pl.program_id(1)

    @pl.when(r < nfull)
    def _():
        out_e_ref[:, :, 0:N] = in_ref[...]
        out_e_ref[:, :, N:N + 1] = clscol_ref[:, :, 0:1]

    @pl.when(r == nfull)
    def _():
        out_e_ref[:, 0:Ed, :] = clsslab_ref[...]

    @pl.when(r == 0)
    def _():
        c = ptr_ref[b + 1] - ptr_ref[b]
        rows = jax.lax.broadcasted_iota(jnp.int32, (1, N + 1, 1), 1)
        cols = jax.lax.broadcasted_iota(jnp.int32, (1, 1, N + 1), 2)
        f_rows = (rows < c) | (rows == N)
        f_cols = (cols < c) | (cols == N)
        out_m_ref[...] = (f_rows & f_cols & (c > 0)).astype(jnp.int8)

    @pl.when((r == 0) & (b == 0))
    def _():
        nrows, ncols = out_nb_ref.shape
        o = (jax.lax.broadcasted_iota(jnp.int32, (nrows, ncols), 0) * ncols
             + jax.lax.broadcasted_iota(jnp.int32, (nrows, ncols), 1))
        nb = jnp.zeros((nrows, ncols), jnp.int32)
        cm = jnp.zeros((nrows, ncols), jnp.bool_)
        for g in range(1, B + 1):
            nb = nb + (o >= ptr_ref[g] + g).astype(jnp.int32)
        for g in range(B):
            cm = cm | (o == ptr_ref[g + 1] + g)
        out_nb_ref[...] = nb
        out_cm_ref[...] = cm.astype(jnp.int8)


def _x_sc_body(params_hbm, x_hbm, cls_hbm, out_hbm,
               pw_v, idx_v, oidx_v, rows_v, cidx_v, bufc, sem):
    cid = lax.axis_index("c")
    sid = lax.axis_index("s")
    wid = sid * 2 + cid

    pltpu.sync_copy(params_hbm.at[wid], pw_v)
    lanes = lax.iota(jnp.int32, 16)
    lo = pw_v[0]
    top = pw_v[1]
    gvec = pw_v[2]

    for k in range(8):
        src = jnp.minimum(lo + (k * 16) + lanes, top)
        idx_v[pl.ds(k * 16, 16)] = src
        oidx_v[pl.ds(k * 16, 16)] = src + gvec
    pltpu.async_copy(x_hbm.at[idx_v], rows_v, sem).wait()
    pltpu.async_copy(rows_v, out_hbm.at[oidx_v], sem).wait()

    @pl.when(wid == 0)
    def _():
        cidx_v[...] = jnp.zeros((16,), jnp.int32)
        pltpu.async_copy(cls_hbm.at[cidx_v], bufc, sem).wait()
        cidx_v[...] = pw_v[3]
        pltpu.async_copy(bufc, out_hbm.at[cidx_v], sem).wait()


def kernel(x, batch_ids, ptr, edges, cls, cls_edge):
    B, N, _, Ed = edges.shape
    T, D = x.shape
    R = N * Ed
    R1 = R + Ed
    TB = T + B
    cols = TB // B
    assert B * cols == TB

    et = jnp.transpose(edges, (0, 1, 3, 2)).reshape(B, R, N)

    clscol = jnp.broadcast_to(
        cls_edge.reshape(1, 1, Ed, 1), (1, _RB // Ed, Ed, 128)
    ).reshape(1, _RB, 128)
    clsslab = jnp.broadcast_to(cls_edge.reshape(1, Ed, 1), (1, Ed, N + 1))

    assert R % _RB == 0
    nfull = R // _RB
    n_rblocks = nfull + 1

    out_e, mask, nb, cm = pl.pallas_call(
        functools.partial(_edges_mask_body, N, Ed, B, nfull),
        grid=(B, n_rblocks),
        in_specs=[
            pl.BlockSpec(memory_space=pltpu.SMEM),
            pl.BlockSpec((1, _RB, N), lambda b, r: (b, jnp.minimum(r, nfull - 1), 0)),
            pl.BlockSpec((1, _RB, 128), lambda b, r: (0, 0, 0)),
            pl.BlockSpec((1, Ed, N + 1), lambda b, r: (0, 0, 0)),
        ],
        out_specs=[
            pl.BlockSpec((1, _RB, N + 1), lambda b, r: (b, r, 0)),
            pl.BlockSpec((1, N + 1, N + 1), lambda b, r: (b, 0, 0)),
            pl.BlockSpec((B, cols), lambda b, r: (0, 0)),
            pl.BlockSpec((B, cols), lambda b, r: (0, 0)),
        ],
        out_shape=[
            jax.ShapeDtypeStruct((B, R1, N + 1), jnp.float32),
            jax.ShapeDtypeStruct((B, N + 1, N + 1), jnp.int8),
            jax.ShapeDtypeStruct((B, cols), jnp.int32),
            jax.ShapeDtypeStruct((B, cols), jnp.int8),
        ],
    )(ptr, et, clscol, clsslab)

    mask = mask.astype(jnp.bool_)

    edges_out = out_e.reshape(B, N + 1, Ed, N + 1).transpose(0, 1, 3, 2)

    nw = 32
    per_g = nw // B
    w = jnp.arange(nw, dtype=jnp.int32)
    gw = w // per_g
    qw = w % per_g
    counts = ptr[1:] - ptr[:-1]
    share = (counts + per_g - 1) // per_g
    lo_w = ptr[gw] + qw * share[gw]
    hi_w = jnp.minimum(lo_w + share[gw], ptr[gw + 1])
    top_w = hi_w - 1
    cls_pos = ptr[1:] + jnp.arange(B, dtype=ptr.dtype)
    cls_row = jnp.concatenate(
        [cls_pos, jnp.broadcast_to(cls_pos[0:1], (16 - B,))])
    params = jnp.stack([
        jnp.broadcast_to(lo_w[:, None], (nw, 16)),
        jnp.broadcast_to(top_w[:, None], (nw, 16)),
        jnp.broadcast_to(gw[:, None], (nw, 16)),
        jnp.broadcast_to(cls_row[None, :], (nw, 16)),
    ], axis=1).astype(jnp.int32)

    mesh = plsc.VectorSubcoreMesh(core_axis_name="c", subcore_axis_name="s")
    new_x = pl.kernel(
        _x_sc_body,
        mesh=mesh,
        out_type=jax.ShapeDtypeStruct((TB, D), x.dtype),
        scratch_types=[
            pltpu.VMEM((4, 16), jnp.int32),
            pltpu.VMEM((128,), jnp.int32),
            pltpu.VMEM((128,), jnp.int32),
            pltpu.VMEM((128, D), jnp.float32),
            pltpu.VMEM((16,), jnp.int32),
            pltpu.VMEM((16, D), jnp.float32),
            pltpu.SemaphoreType.DMA,
        ],
    )(params, x, cls.reshape(1, D))

    new_batch = nb.reshape(TB)
    cls_mask = cm.reshape(TB).astype(jnp.bool_)
    new_ptr = ptr + jnp.arange(B + 1, dtype=ptr.dtype)
    return new_x, mask, edges_out, cls_mask, new_batch, new_ptr


# speedup vs baseline: 1.0949x; 1.0949x over previous
import functools

import jax
import jax.numpy as jnp
from jax import lax
from jax.experimental import pallas as pl
from jax.experimental.pallas import tpu as pltpu
from jax.experimental.pallas import tpu_sc as plsc


_RB = 4096
_CH = 16


def _edges_mask_body(N, Ed, B, nfull, ptr_ref, in_ref, clscol_ref, clsslab_ref,
                     out_e_ref, out_m_ref, out_nb_ref, out_cm_ref):
    b = pl.program_id(0)
    r = pl.program_id(1)

    @pl.when(r < nfull)
    def _():
        out_e_ref[:, :, 0:N] = in_ref[...]
        out_e_ref[:, :, N:N + 1] = clscol_ref[:, :, 0:1]

    @pl.when(r == nfull)
    def _():
        out_e_ref[:, 0:Ed, :] = clsslab_ref[...]

    @pl.when(r == 0)
    def _():
        c = ptr_ref[b + 1] - ptr_ref[b]
        rows = jax.lax.broadcasted_iota(jnp.int32, (1, N + 1, 1), 1)
        cols = jax.lax.broadcasted_iota(jnp.int32, (1, 1, N + 1), 2)
        f_rows = (rows < c) | (rows == N)
        f_cols = (cols < c) | (cols == N)
        out_m_ref[...] = (f_rows & f_cols & (c > 0)).astype(jnp.int8)

    @pl.when((r == 0) & (b == 0))
    def _():
        nrows, ncols = out_nb_ref.shape
        o = (jax.lax.broadcasted_iota(jnp.int32, (nrows, ncols), 0) * ncols
             + jax.lax.broadcasted_iota(jnp.int32, (nrows, ncols), 1))
        nb = jnp.zeros((nrows, ncols), jnp.int32)
        cm = jnp.zeros((nrows, ncols), jnp.bool_)
        for g in range(1, B + 1):
            nb = nb + (o >= ptr_ref[g] + g).astype(jnp.int32)
        for g in range(B):
            cm = cm | (o == ptr_ref[g + 1] + g)
        out_nb_ref[...] = nb
        out_cm_ref[...] = cm.astype(jnp.int8)


def _x_sc_body(params_hbm, x_hbm, cls_hbm, out_hbm,
               pw_v, idx_a, idx_b, oidx_a, oidx_b, rows_a, rows_b,
               cidx_v, bufc, sem_a, sem_b, sem_c):
    cid = lax.axis_index("c")
    sid = lax.axis_index("s")
    wid = sid * 2 + cid

    @pl.when(wid == 0)
    def _():
        cidx_v[...] = jnp.zeros((16,), jnp.int32)
        pltpu.async_copy(cls_hbm.at[cidx_v], bufc, sem_c)

    pltpu.sync_copy(params_hbm.at[wid], pw_v)
    lanes = lax.iota(jnp.int32, 16)
    lo = pw_v[0]
    top = pw_v[1]
    gvec = pw_v[2]

    for k in range(4):
        src = jnp.minimum(lo + (k * 16) + lanes, top)
        idx_a[pl.ds(k * 16, 16)] = src
        oidx_a[pl.ds(k * 16, 16)] = src + gvec
    for k in range(4):
        src = jnp.minimum(lo + ((k + 4) * 16) + lanes, top)
        idx_b[pl.ds(k * 16, 16)] = src
        oidx_b[pl.ds(k * 16, 16)] = src + gvec

    ga = pltpu.async_copy(x_hbm.at[idx_a], rows_a, sem_a)
    gb = pltpu.async_copy(x_hbm.at[idx_b], rows_b, sem_b)
    ga.wait()
    sa = pltpu.async_copy(rows_a, out_hbm.at[oidx_a], sem_a)
    gb.wait()
    sb = pltpu.async_copy(rows_b, out_hbm.at[oidx_b], sem_b)
    sa.wait()
    sb.wait()

    @pl.when(wid == 0)
    def _():
        pltpu.make_async_copy(cls_hbm.at[cidx_v], bufc, sem_c).wait()
        cidx_v[...] = pw_v[3]
        pltpu.async_copy(bufc, out_hbm.at[cidx_v], sem_c).wait()


def kernel(x, batch_ids, ptr, edges, cls, cls_edge):
    B, N, _, Ed = edges.shape
    T, D = x.shape
    R = N * Ed
    R1 = R + Ed
    TB = T + B
    cols = TB // B
    assert B * cols == TB

    et = jnp.transpose(edges, (0, 1, 3, 2)).reshape(B, R, N)

    clscol = jnp.broadcast_to(
        cls_edge.reshape(1, 1, Ed, 1), (1, _RB // Ed, Ed, 128)
    ).reshape(1, _RB, 128)
    clsslab = jnp.broadcast_to(cls_edge.reshape(1, Ed, 1), (1, Ed, N + 1))

    assert R % _RB == 0
    nfull = R // _RB
    n_rblocks = nfull + 1

    out_e, mask, nb, cm = pl.pallas_call(
        functools.partial(_edges_mask_body, N, Ed, B, nfull),
        grid=(B, n_rblocks),
        in_specs=[
            pl.BlockSpec(memory_space=pltpu.SMEM),
            pl.BlockSpec((1, _RB, N), lambda b, r: (b, jnp.minimum(r, nfull - 1), 0)),
            pl.BlockSpec((1, _RB, 128), lambda b, r: (0, 0, 0)),
            pl.BlockSpec((1, Ed, N + 1), lambda b, r: (0, 0, 0)),
        ],
        out_specs=[
            pl.BlockSpec((1, _RB, N + 1), lambda b, r: (b, r, 0)),
            pl.BlockSpec((1, N + 1, N + 1), lambda b, r: (b, 0, 0)),
            pl.BlockSpec((B, cols), lambda b, r: (0, 0)),
            pl.BlockSpec((B, cols), lambda b, r: (0, 0)),
        ],
        out_shape=[
            jax.ShapeDtypeStruct((B, R1, N + 1), jnp.float32),
            jax.ShapeDtypeStruct((B, N + 1, N + 1), jnp.int8),
            jax.ShapeDtypeStruct((B, cols), jnp.int32),
            jax.ShapeDtypeStruct((B, cols), jnp.int8),
        ],
    )(ptr, et, clscol, clsslab)

    mask = mask.astype(jnp.bool_)

    edges_out = out_e.reshape(B, N + 1, Ed, N + 1).transpose(0, 1, 3, 2)

    nw = 32
    per_g = nw // B
    w = jnp.arange(nw, dtype=jnp.int32)
    gw = w // per_g
    qw = w % per_g
    counts = ptr[1:] - ptr[:-1]
    share = (counts + per_g - 1) // per_g
    lo_w = ptr[gw] + qw * share[gw]
    hi_w = jnp.minimum(lo_w + share[gw], ptr[gw + 1])
    top_w = hi_w - 1
    cls_pos = ptr[1:] + jnp.arange(B, dtype=ptr.dtype)
    cls_row = jnp.concatenate(
        [cls_pos, jnp.broadcast_to(cls_pos[0:1], (16 - B,))])
    params = jnp.stack([
        jnp.broadcast_to(lo_w[:, None], (nw, 16)),
        jnp.broadcast_to(top_w[:, None], (nw, 16)),
        jnp.broadcast_to(gw[:, None], (nw, 16)),
        jnp.broadcast_to(cls_row[None, :], (nw, 16)),
    ], axis=1).astype(jnp.int32)

    mesh = plsc.VectorSubcoreMesh(core_axis_name="c", subcore_axis_name="s")
    new_x = pl.kernel(
        _x_sc_body,
        mesh=mesh,
        out_type=jax.ShapeDtypeStruct((TB, D), x.dtype),
        scratch_types=[
            pltpu.VMEM((4, 16), jnp.int32),
            pltpu.VMEM((64,), jnp.int32),
            pltpu.VMEM((64,), jnp.int32),
            pltpu.VMEM((64,), jnp.int32),
            pltpu.VMEM((64,), jnp.int32),
            pltpu.VMEM((64, D), jnp.float32),
            pltpu.VMEM((64, D), jnp.float32),
            pltpu.VMEM((16,), jnp.int32),
            pltpu.VMEM((16, D), jnp.float32),
            pltpu.SemaphoreType.DMA,
            pltpu.SemaphoreType.DMA,
            pltpu.SemaphoreType.DMA,
        ],
    )(params, x, cls.reshape(1, D))

    new_batch = nb.reshape(TB)
    cls_mask = cm.reshape(TB).astype(jnp.bool_)
    new_ptr = ptr + jnp.arange(B + 1, dtype=ptr.dtype)
    return new_x, mask, edges_out, cls_mask, new_batch, new_ptr


# speedup vs baseline: 1.0979x; 1.0028x over previous
import functools

import jax
import jax.numpy as jnp
from jax import lax
from jax.experimental import pallas as pl
from jax.experimental.pallas import tpu as pltpu
from jax.experimental.pallas import tpu_sc as plsc


_RB = 4096
_CH = 16


def _bk_body(B, ptr_ref, out_nb_ref, out_cm_ref):
    nrows, ncols = out_nb_ref.shape
    o = (jax.lax.broadcasted_iota(jnp.int32, (nrows, ncols), 0) * ncols
         + jax.lax.broadcasted_iota(jnp.int32, (nrows, ncols), 1))
    nb = jnp.zeros((nrows, ncols), jnp.int32)
    cm = jnp.zeros((nrows, ncols), jnp.bool_)
    for g in range(1, B + 1):
        nb = nb + (o >= ptr_ref[g] + g).astype(jnp.int32)
    for g in range(B):
        cm = cm | (o == ptr_ref[g + 1] + g)
    out_nb_ref[...] = nb
    out_cm_ref[...] = cm.astype(jnp.int8)


def _edges_mask_body(N, Ed, B, nfull, ptr_ref, in_ref, clscol_ref, clsslab_ref,
                     out_e_ref, out_m_ref):
    b = pl.program_id(0)
    r = pl.program_id(1)

    @pl.when(r < nfull)
    def _():
        out_e_ref[:, :, 0:N] = in_ref[...]
        out_e_ref[:, :, N:N + 1] = clscol_ref[:, :, 0:1]

    @pl.when(r == nfull)
    def _():
        out_e_ref[:, 0:Ed, :] = clsslab_ref[...]

    @pl.when(r == 0)
    def _():
        c = ptr_ref[b + 1] - ptr_ref[b]
        rows = jax.lax.broadcasted_iota(jnp.int32, (1, N + 1, 1), 1)
        cols = jax.lax.broadcasted_iota(jnp.int32, (1, 1, N + 1), 2)
        f_rows = (rows < c) | (rows == N)
        f_cols = (cols < c) | (cols == N)
        out_m_ref[...] = (f_rows & f_cols & (c > 0)).astype(jnp.int8)


def _x_sc_body(params_hbm, x_hbm, cls_hbm, out_hbm,
               pw_v, idx_a, idx_b, oidx_a, oidx_b, rows_a, rows_b,
               cidx_v, bufc, sem_a, sem_b, sem_c):
    cid = lax.axis_index("c")
    sid = lax.axis_index("s")
    wid = sid * 2 + cid

    @pl.when(wid == 0)
    def _():
        cidx_v[...] = jnp.zeros((16,), jnp.int32)
        pltpu.async_copy(cls_hbm.at[cidx_v], bufc, sem_c)

    pltpu.sync_copy(params_hbm.at[wid], pw_v)
    lanes = lax.iota(jnp.int32, 16)
    lo = pw_v[0]
    top = pw_v[1]
    gvec = pw_v[2]

    for k in range(4):
        src = jnp.minimum(lo + (k * 16) + lanes, top)
        idx_a[pl.ds(k * 16, 16)] = src
        oidx_a[pl.ds(k * 16, 16)] = src + gvec
    for k in range(4):
        src = jnp.minimum(lo + ((k + 4) * 16) + lanes, top)
        idx_b[pl.ds(k * 16, 16)] = src
        oidx_b[pl.ds(k * 16, 16)] = src + gvec

    ga = pltpu.async_copy(x_hbm.at[idx_a], rows_a, sem_a)
    gb = pltpu.async_copy(x_hbm.at[idx_b], rows_b, sem_b)
    ga.wait()
    sa = pltpu.async_copy(rows_a, out_hbm.at[oidx_a], sem_a)
    gb.wait()
    sb = pltpu.async_copy(rows_b, out_hbm.at[oidx_b], sem_b)
    sa.wait()
    sb.wait()

    @pl.when(wid == 0)
    def _():
        pltpu.make_async_copy(cls_hbm.at[cidx_v], bufc, sem_c).wait()
        cidx_v[...] = pw_v[3]
        pltpu.async_copy(bufc, out_hbm.at[cidx_v], sem_c).wait()


def kernel(x, batch_ids, ptr, edges, cls, cls_edge):
    B, N, _, Ed = edges.shape
    T, D = x.shape
    R = N * Ed
    R1 = R + Ed
    TB = T + B
    cols = TB // B
    assert B * cols == TB

    et = jnp.transpose(edges, (0, 1, 3, 2)).reshape(B, R, N)

    clscol = jnp.broadcast_to(
        cls_edge.reshape(1, 1, Ed, 1), (1, _RB // Ed, Ed, 128)
    ).reshape(1, _RB, 128)
    clsslab = jnp.broadcast_to(cls_edge.reshape(1, Ed, 1), (1, Ed, N + 1))

    assert R % _RB == 0
    nfull = R // _RB
    n_rblocks = nfull + 1

    out_e, mask = pl.pallas_call(
        functools.partial(_edges_mask_body, N, Ed, B, nfull),
        grid=(B, n_rblocks),
        in_specs=[
            pl.BlockSpec(memory_space=pltpu.SMEM),
            pl.BlockSpec((1, _RB, N), lambda b, r: (b, jnp.minimum(r, nfull - 1), 0)),
            pl.BlockSpec((1, _RB, 128), lambda b, r: (0, 0, 0)),
            pl.BlockSpec((1, Ed, N + 1), lambda b, r: (0, 0, 0)),
        ],
        out_specs=[
            pl.BlockSpec((1, _RB, N + 1), lambda b, r: (b, r, 0)),
            pl.BlockSpec((1, N + 1, N + 1), lambda b, r: (b, 0, 0)),
        ],
        out_shape=[
            jax.ShapeDtypeStruct((B, R1, N + 1), jnp.float32),
            jax.ShapeDtypeStruct((B, N + 1, N + 1), jnp.int8),
        ],
    )(ptr, et, clscol, clsslab)

    nb, cm = pl.pallas_call(
        functools.partial(_bk_body, B),
        in_specs=[pl.BlockSpec(memory_space=pltpu.SMEM)],
        out_specs=[
            pl.BlockSpec(memory_space=pltpu.VMEM),
            pl.BlockSpec(memory_space=pltpu.VMEM),
        ],
        out_shape=[
            jax.ShapeDtypeStruct((B, cols), jnp.int32),
            jax.ShapeDtypeStruct((B, cols), jnp.int8),
        ],
    )(ptr)

    mask = mask.astype(jnp.bool_)

    edges_out = out_e.reshape(B, N + 1, Ed, N + 1).transpose(0, 1, 3, 2)

    nw = 32
    per_g = nw // B
    w = jnp.arange(nw, dtype=jnp.int32)
    gw = w // per_g
    qw = w % per_g
    counts = ptr[1:] - ptr[:-1]
    share = (counts + per_g - 1) // per_g
    lo_w = ptr[gw] + qw * share[gw]
    hi_w = jnp.minimum(lo_w + share[gw], ptr[gw + 1])
    top_w = hi_w - 1
    cls_pos = ptr[1:] + jnp.arange(B, dtype=ptr.dtype)
    cls_row = jnp.concatenate(
        [cls_pos, jnp.broadcast_to(cls_pos[0:1], (16 - B,))])
    params = jnp.stack([
        jnp.broadcast_to(lo_w[:, None], (nw, 16)),
        jnp.broadcast_to(top_w[:, None], (nw, 16)),
        jnp.broadcast_to(gw[:, None], (nw, 16)),
        jnp.broadcast_to(cls_row[None, :], (nw, 16)),
    ], axis=1).astype(jnp.int32)

    mesh = plsc.VectorSubcoreMesh(core_axis_name="c", subcore_axis_name="s")
    new_x = pl.kernel(
        _x_sc_body,
        mesh=mesh,
        out_type=jax.ShapeDtypeStruct((TB, D), x.dtype),
        scratch_types=[
            pltpu.VMEM((4, 16), jnp.int32),
            pltpu.VMEM((64,), jnp.int32),
            pltpu.VMEM((64,), jnp.int32),
            pltpu.VMEM((64,), jnp.int32),
            pltpu.VMEM((64,), jnp.int32),
            pltpu.VMEM((64, D), jnp.float32),
            pltpu.VMEM((64, D), jnp.float32),
            pltpu.VMEM((16,), jnp.int32),
            pltpu.VMEM((16, D), jnp.float32),
            pltpu.SemaphoreType.DMA,
            pltpu.SemaphoreType.DMA,
            pltpu.SemaphoreType.DMA,
        ],
    )(params, x, cls.reshape(1, D))

    new_batch = nb.reshape(TB)
    cls_mask = cm.reshape(TB).astype(jnp.bool_)
    new_ptr = ptr + jnp.arange(B + 1, dtype=ptr.dtype)
    return new_x, mask, edges_out, cls_mask, new_batch, new_ptr
